# baseline (speedup 1.0000x reference)
import functools

import jax
import jax.numpy as jnp
from jax import lax
from jax.experimental import pallas as pl
from jax.experimental.pallas import tpu as pltpu
from jax.experimental.pallas import tpu_sc as plsc

N_NODES = 10000
N_PAD = 10240
N_EDGES = 320000
NC = 2
NS = 16
NW = NC * NS
K = 80
NITER = 126
E_PAD = NW * NITER * K
ROWS_PT = N_PAD // NS
DEGW = 16

_mesh = plsc.VectorSubcoreMesh(core_axis_name="c", subcore_axis_name="s")


def _deg_body(dst_hbm, ones_hbm, zeros_hbm, out_hbm, deg_sp, idx_v, ones_v):
    c = lax.axis_index("c")
    s = lax.axis_index("s")
    wid = s * NC + c
    pltpu.sync_copy(zeros_hbm.at[pl.ds(s * ROWS_PT, ROWS_PT)],
                    deg_sp.at[pl.ds(s * ROWS_PT, ROWS_PT)])
    pltpu.sync_copy(ones_hbm, ones_v)
    pltpu.sync_copy(dst_hbm.at[wid], idx_v)
    plsc.subcore_barrier()

    def chunk(j, carry):
        pltpu.sync_copy(ones_v, deg_sp.at[idx_v.at[j]], add=True)
        return carry

    lax.fori_loop(0, NITER, chunk, 0, unroll=False)
    plsc.subcore_barrier()
    pltpu.sync_copy(deg_sp.at[pl.ds(s * ROWS_PT, ROWS_PT)],
                    out_hbm.at[c, pl.ds(s * ROWS_PT, ROWS_PT)])


def _degree_partials(dst_r, ones16, zeros16):
    return pl.kernel(
        _deg_body,
        out_type=jax.ShapeDtypeStruct((NC, N_PAD, DEGW), jnp.float32),
        mesh=_mesh,
        scratch_types=[
            pltpu.VMEM_SHARED((N_PAD, DEGW), jnp.float32),
            pltpu.VMEM((NITER, K), jnp.int32),
            pltpu.VMEM((K, DEGW), jnp.float32),
        ],
        compiler_params=pltpu.CompilerParams(use_tc_tiling_on_sc=False),
    )(dst_r, ones16, zeros16)


def _scatter_body(src_hbm, dst_hbm, u_hbm, zeros_hbm, out_hbm,
                  agg_sp, sidx_v, didx_v, rows_v, sem):
    c = lax.axis_index("c")
    s = lax.axis_index("s")
    wid = s * NC + c
    pltpu.sync_copy(zeros_hbm.at[pl.ds(s * ROWS_PT, ROWS_PT)],
                    agg_sp.at[pl.ds(s * ROWS_PT, ROWS_PT)])
    pltpu.sync_copy(src_hbm.at[wid], sidx_v)
    pltpu.sync_copy(dst_hbm.at[wid], didx_v)
    plsc.subcore_barrier()

    pltpu.async_copy(u_hbm.at[sidx_v.at[0]], rows_v.at[0], sem.at[0])
    pltpu.async_copy(u_hbm.at[sidx_v.at[1]], rows_v.at[1], sem.at[1])

    def chunk(j, carry):
        b = lax.rem(j, 2)
        pltpu.make_async_copy(u_hbm.at[sidx_v.at[j]], rows_v.at[b],
                              sem.at[b]).wait()
        pltpu.sync_copy(rows_v.at[b], agg_sp.at[didx_v.at[j]], add=True)

        @pl.when(j + 2 < NITER)
        def _():
            pltpu.async_copy(u_hbm.at[sidx_v.at[j + 2]], rows_v.at[b],
                             sem.at[b])

        return carry

    lax.fori_loop(0, NITER, chunk, 0, unroll=False)
    plsc.subcore_barrier()
    pltpu.sync_copy(agg_sp.at[pl.ds(s * ROWS_PT, ROWS_PT)],
                    out_hbm.at[c, pl.ds(s * ROWS_PT, ROWS_PT)])


def _scatter_partials(src_r, dst_r, u, zeros_d, d):
    return pl.kernel(
        _scatter_body,
        out_type=jax.ShapeDtypeStruct((NC, N_PAD, d), jnp.float32),
        mesh=_mesh,
        scratch_types=[
            pltpu.VMEM_SHARED((N_PAD, d), jnp.float32),
            pltpu.VMEM((NITER, K), jnp.int32),
            pltpu.VMEM((NITER, K), jnp.int32),
            pltpu.VMEM((2, K, d), jnp.float32),
            pltpu.SemaphoreType.DMA((2,)),
        ],
        compiler_params=pltpu.CompilerParams(use_tc_tiling_on_sc=False),
    )(src_r, dst_r, u, zeros_d)


_RB = 1000


def _dinv_of(degp_ref):
    deg = degp_ref[0, :, 0:1] + degp_ref[1, :, 0:1] + 1.0
    return lax.rsqrt(deg)


def _mm1_body(degp_ref, x_ref, w_ref, u_ref):
    dinv = _dinv_of(degp_ref)
    xw = jnp.dot(x_ref[...], w_ref[...], preferred_element_type=jnp.float32)
    u_ref[...] = dinv * xw


def _mid_body(degp_ref, agg_ref, u_ref, b_ref, w_ref, u2_ref):
    dinv = _dinv_of(degp_ref)
    tot = agg_ref[0] + agg_ref[1] + u_ref[...]
    h = jnp.maximum(dinv * tot + b_ref[...], 0.0)
    hw = jnp.dot(h, w_ref[...], preferred_element_type=jnp.float32)
    u2_ref[...] = dinv * hw


def _fin_body(degp_ref, agg_ref, u_ref, b_ref, out_ref):
    dinv = _dinv_of(degp_ref)
    out_ref[...] = dinv * (agg_ref[0] + agg_ref[1] + u_ref[...]) + b_ref[...]


def _deg_spec():
    return pl.BlockSpec((NC, _RB, DEGW), lambda i: (0, i, 0))


def _full(shape):
    return pl.BlockSpec(shape, lambda i: tuple(0 for _ in shape))


def _tc_mm1(degp, x, w1):
    return pl.pallas_call(
        _mm1_body,
        grid=(N_NODES // _RB,),
        in_specs=[_deg_spec(),
                  pl.BlockSpec((_RB, 128), lambda i: (i, 0)),
                  _full((128, 128))],
        out_specs=pl.BlockSpec((_RB, 128), lambda i: (i, 0)),
        out_shape=jax.ShapeDtypeStruct((N_NODES, 128), jnp.float32),
    )(degp, x, w1)


def _tc_mid(degp, agg1, u1, b1, w2):
    return pl.pallas_call(
        _mid_body,
        grid=(N_NODES // _RB,),
        in_specs=[_deg_spec(),
                  pl.BlockSpec((NC, _RB, 128), lambda i: (0, i, 0)),
                  pl.BlockSpec((_RB, 128), lambda i: (i, 0)),
                  _full((1, 128)),
                  _full((128, 64))],
        out_specs=pl.BlockSpec((_RB, 64), lambda i: (i, 0)),
        out_shape=jax.ShapeDtypeStruct((N_NODES, 64), jnp.float32),
    )(degp, agg1, u1, b1, w2)


def _tc_fin(degp, agg2, u2, b2):
    return pl.pallas_call(
        _fin_body,
        grid=(N_NODES // _RB,),
        in_specs=[_deg_spec(),
                  pl.BlockSpec((NC, _RB, 64), lambda i: (0, i, 0)),
                  pl.BlockSpec((_RB, 64), lambda i: (i, 0)),
                  _full((1, 64))],
        out_specs=pl.BlockSpec((_RB, 64), lambda i: (i, 0)),
        out_shape=jax.ShapeDtypeStruct((N_NODES, 64), jnp.float32),
    )(degp, agg2, u2, b2)


def kernel(x, edge_index, W1, b1, W2, b2):
    npad = E_PAD - N_EDGES
    pad_dst = N_NODES + (jnp.arange(npad, dtype=jnp.int32)
                         % (N_PAD - N_NODES))
    src_r = jnp.concatenate(
        [edge_index[0].astype(jnp.int32),
         jnp.zeros((npad,), jnp.int32)]).reshape(NW, NITER, K)
    dst_r = jnp.concatenate(
        [edge_index[1].astype(jnp.int32), pad_dst]).reshape(NW, NITER, K)
    ones16 = jnp.ones((K, DEGW), jnp.float32)
    zeros16 = jnp.zeros((N_PAD, DEGW), jnp.float32)
    zeros128 = jnp.zeros((N_PAD, 128), jnp.float32)
    zeros64 = jnp.zeros((N_PAD, 64), jnp.float32)

    degp = _degree_partials(dst_r, ones16, zeros16)
    u1 = _tc_mm1(degp, x, W1)
    agg1 = _scatter_partials(src_r, dst_r, u1, zeros128, 128)
    u2 = _tc_mid(degp, agg1, u1, b1.reshape(1, 128), W2)
    agg2 = _scatter_partials(src_r, dst_r, u2, zeros64, 64)
    return _tc_fin(degp, agg2, u2, b2.reshape(1, 64))


# speedup vs baseline: 1.5816x; 1.5816x over previous
import functools

import jax
import jax.numpy as jnp
from jax import lax
from jax.experimental import pallas as pl
from jax.experimental.pallas import tpu as pltpu
from jax.experimental.pallas import tpu_sc as plsc

N_NODES = 10000
N_PAD = 10240
N_EDGES = 320000
NC = 2
NS = 16
NW = NC * NS
K = 80
NITER = 126
E_PAD = NW * NITER * K
ROWS_PT = N_PAD // NS
DEGW = 16

_mesh = plsc.VectorSubcoreMesh(core_axis_name="c", subcore_axis_name="s")


def _deg_body(dst_hbm, ones_hbm, zeros_hbm, out_hbm, deg_sp, idx_v, ones_v):
    c = lax.axis_index("c")
    s = lax.axis_index("s")
    wid = s * NC + c
    pltpu.sync_copy(zeros_hbm.at[pl.ds(s * ROWS_PT, ROWS_PT)],
                    deg_sp.at[pl.ds(s * ROWS_PT, ROWS_PT)])
    pltpu.sync_copy(ones_hbm, ones_v)
    pltpu.sync_copy(dst_hbm.at[wid], idx_v)
    plsc.subcore_barrier()

    def chunk(j, carry):
        pltpu.sync_copy(ones_v, deg_sp.at[idx_v.at[j]], add=True)
        return carry

    lax.fori_loop(0, NITER, chunk, 0, unroll=False)
    plsc.subcore_barrier()
    pltpu.sync_copy(deg_sp.at[pl.ds(s * ROWS_PT, ROWS_PT)],
                    out_hbm.at[c, pl.ds(s * ROWS_PT, ROWS_PT)])


def _degree_partials(dst_r, ones16, zeros16):
    return pl.kernel(
        _deg_body,
        out_type=jax.ShapeDtypeStruct((NC, N_PAD, DEGW), jnp.float32),
        mesh=_mesh,
        scratch_types=[
            pltpu.VMEM_SHARED((N_PAD, DEGW), jnp.float32),
            pltpu.VMEM((NITER, K), jnp.int32),
            pltpu.VMEM((K, DEGW), jnp.float32),
        ],
        compiler_params=pltpu.CompilerParams(use_tc_tiling_on_sc=False),
    )(dst_r, ones16, zeros16)


def _scatter_body(src_hbm, dst_hbm, u_hbm, zeros_hbm, out_hbm,
                  agg_sp, sidx_v, didx_v, rows_v, sem):
    c = lax.axis_index("c")
    s = lax.axis_index("s")
    wid = s * NC + c
    pltpu.sync_copy(zeros_hbm.at[pl.ds(s * ROWS_PT, ROWS_PT)],
                    agg_sp.at[pl.ds(s * ROWS_PT, ROWS_PT)])
    pltpu.sync_copy(src_hbm.at[wid], sidx_v)
    pltpu.sync_copy(dst_hbm.at[wid], didx_v)
    plsc.subcore_barrier()

    pltpu.async_copy(u_hbm.at[sidx_v.at[0]], rows_v.at[0], sem.at[0])
    pltpu.async_copy(u_hbm.at[sidx_v.at[1]], rows_v.at[1], sem.at[1])

    def chunk(j, carry):
        b = lax.rem(j, 2)
        pltpu.make_async_copy(u_hbm.at[sidx_v.at[j]], rows_v.at[b],
                              sem.at[b]).wait()
        pltpu.sync_copy(rows_v.at[b], agg_sp.at[didx_v.at[j]], add=True)

        @pl.when(j + 2 < NITER)
        def _():
            pltpu.async_copy(u_hbm.at[sidx_v.at[j + 2]], rows_v.at[b],
                             sem.at[b])

        return carry

    lax.fori_loop(0, NITER, chunk, 0, unroll=False)
    plsc.subcore_barrier()
    pltpu.sync_copy(agg_sp.at[pl.ds(s * ROWS_PT, ROWS_PT)],
                    out_hbm.at[c, pl.ds(s * ROWS_PT, ROWS_PT)])


def _scatter_partials(src_r, dst_r, u, zeros_d, d):
    return pl.kernel(
        _scatter_body,
        out_type=jax.ShapeDtypeStruct((NC, N_PAD, d), jnp.float32),
        mesh=_mesh,
        scratch_types=[
            pltpu.VMEM_SHARED((N_PAD, d), jnp.float32),
            pltpu.VMEM((NITER, K), jnp.int32),
            pltpu.VMEM((NITER, K), jnp.int32),
            pltpu.VMEM((2, K, d), jnp.float32),
            pltpu.SemaphoreType.DMA((2,)),
        ],
        compiler_params=pltpu.CompilerParams(use_tc_tiling_on_sc=False),
    )(src_r, dst_r, u, zeros_d)


_RB = 1000


def _dinv_of(degp_ref):
    deg = degp_ref[0, :, 0:1] + degp_ref[1, :, 0:1] + 1.0
    return lax.rsqrt(deg)


def _mm1_body(degp_ref, x_ref, w_ref, u_ref):
    dinv = _dinv_of(degp_ref)
    xw = jnp.dot(x_ref[...], w_ref[...], preferred_element_type=jnp.float32)
    u_ref[...] = dinv * xw


def _mid_body(degp_ref, agg_ref, u_ref, b_ref, w_ref, u2_ref):
    dinv = _dinv_of(degp_ref)
    tot = agg_ref[0] + agg_ref[1] + u_ref[...]
    h = jnp.maximum(dinv * tot + b_ref[...], 0.0)
    hw = jnp.dot(h, w_ref[...], preferred_element_type=jnp.float32)
    u2_ref[...] = dinv * hw


def _fin_body(degp_ref, agg_ref, u_ref, b_ref, out_ref):
    dinv = _dinv_of(degp_ref)
    out_ref[...] = dinv * (agg_ref[0] + agg_ref[1] + u_ref[...]) + b_ref[...]


def _deg_spec():
    return pl.BlockSpec((NC, _RB, DEGW), lambda i: (0, i, 0))


def _full(shape):
    return pl.BlockSpec(shape, lambda i: tuple(0 for _ in shape))


def _tc_mm1(degp, x, w1):
    return pl.pallas_call(
        _mm1_body,
        grid=(N_NODES // _RB,),
        in_specs=[_deg_spec(),
                  pl.BlockSpec((_RB, 128), lambda i: (i, 0)),
                  _full((128, 128))],
        out_specs=pl.BlockSpec((_RB, 128), lambda i: (i, 0)),
        out_shape=jax.ShapeDtypeStruct((N_NODES, 128), jnp.float32),
    )(degp, x, w1)


def _tc_mid(degp, agg1, u1, b1, w2):
    return pl.pallas_call(
        _mid_body,
        grid=(N_NODES // _RB,),
        in_specs=[_deg_spec(),
                  pl.BlockSpec((NC, _RB, 128), lambda i: (0, i, 0)),
                  pl.BlockSpec((_RB, 128), lambda i: (i, 0)),
                  _full((1, 128)),
                  _full((128, 64))],
        out_specs=pl.BlockSpec((_RB, 64), lambda i: (i, 0)),
        out_shape=jax.ShapeDtypeStruct((N_NODES, 64), jnp.float32),
    )(degp, agg1, u1, b1, w2)


def _tc_fin(degp, agg2, u2, b2):
    return pl.pallas_call(
        _fin_body,
        grid=(N_NODES // _RB,),
        in_specs=[_deg_spec(),
                  pl.BlockSpec((NC, _RB, 64), lambda i: (0, i, 0)),
                  pl.BlockSpec((_RB, 64), lambda i: (i, 0)),
                  _full((1, 64))],
        out_specs=pl.BlockSpec((_RB, 64), lambda i: (i, 0)),
        out_shape=jax.ShapeDtypeStruct((N_NODES, 64), jnp.float32),
    )(degp, agg2, u2, b2)


def kernel(x, edge_index, W1, b1, W2, b2):
    npad = E_PAD - N_EDGES
    pad_iota = jnp.arange(npad, dtype=jnp.int32)
    pad_src = (pad_iota * 7) % N_NODES
    pad_dst = N_NODES + pad_iota % (N_PAD - N_NODES)
    src_r = jnp.concatenate(
        [edge_index[0].astype(jnp.int32), pad_src]).reshape(NW, NITER, K)
    dst_r = jnp.concatenate(
        [edge_index[1].astype(jnp.int32), pad_dst]).reshape(NW, NITER, K)
    ones16 = jnp.ones((K, DEGW), jnp.float32)
    zeros16 = jnp.zeros((N_PAD, DEGW), jnp.float32)
    zeros128 = jnp.zeros((N_PAD, 128), jnp.float32)
    zeros64 = jnp.zeros((N_PAD, 64), jnp.float32)

    degp = _degree_partials(dst_r, ones16, zeros16)
    u1 = _tc_mm1(degp, x, W1)
    agg1 = _scatter_partials(src_r, dst_r, u1, zeros128, 128)
    u2 = _tc_mid(degp, agg1, u1, b1.reshape(1, 128), W2)
    agg2 = _scatter_partials(src_r, dst_r, u2, zeros64, 64)
    return _tc_fin(degp, agg2, u2, b2.reshape(1, 64))


# speedup vs baseline: 1.7070x; 1.0793x over previous
import functools

import jax
import jax.numpy as jnp
from jax import lax
from jax.experimental import pallas as pl
from jax.experimental.pallas import tpu as pltpu
from jax.experimental.pallas import tpu_sc as plsc

N_NODES = 10000
N_PAD = 10240
N_EDGES = 320000
NC = 2
NS = 16
NW = NC * NS
K = 112
NITER = 90
E_PAD = NW * NITER * K
ROWS_PT = N_PAD // NS
DEGW = 16

_mesh = plsc.VectorSubcoreMesh(core_axis_name="c", subcore_axis_name="s")


def _deg_body(dst_hbm, ones_hbm, zeros_hbm, out_hbm, deg_sp, idx_v, ones_v):
    c = lax.axis_index("c")
    s = lax.axis_index("s")
    wid = s * NC + c
    pltpu.sync_copy(zeros_hbm.at[pl.ds(s * ROWS_PT, ROWS_PT)],
                    deg_sp.at[pl.ds(s * ROWS_PT, ROWS_PT)])
    pltpu.sync_copy(ones_hbm, ones_v)
    pltpu.sync_copy(dst_hbm.at[wid], idx_v)
    plsc.subcore_barrier()

    def chunk(j, carry):
        pltpu.sync_copy(ones_v, deg_sp.at[idx_v.at[j]], add=True)
        return carry

    lax.fori_loop(0, NITER, chunk, 0, unroll=False)
    plsc.subcore_barrier()
    pltpu.sync_copy(deg_sp.at[pl.ds(s * ROWS_PT, ROWS_PT)],
                    out_hbm.at[c, pl.ds(s * ROWS_PT, ROWS_PT)])


def _degree_partials(dst_r, ones16, zeros16):
    return pl.kernel(
        _deg_body,
        out_type=jax.ShapeDtypeStruct((NC, N_PAD, DEGW), jnp.float32),
        mesh=_mesh,
        scratch_types=[
            pltpu.VMEM_SHARED((N_PAD, DEGW), jnp.float32),
            pltpu.VMEM((NITER, K), jnp.int32),
            pltpu.VMEM((K, DEGW), jnp.float32),
        ],
        compiler_params=pltpu.CompilerParams(use_tc_tiling_on_sc=False),
    )(dst_r, ones16, zeros16)


def _scatter_body(src_hbm, dst_hbm, u_hbm, zeros_hbm, out_hbm,
                  agg_sp, sidx_v, didx_v, rows_v, sem):
    c = lax.axis_index("c")
    s = lax.axis_index("s")
    wid = s * NC + c
    pltpu.sync_copy(zeros_hbm.at[pl.ds(s * ROWS_PT, ROWS_PT)],
                    agg_sp.at[pl.ds(s * ROWS_PT, ROWS_PT)])
    pltpu.sync_copy(src_hbm.at[wid], sidx_v)
    pltpu.sync_copy(dst_hbm.at[wid], didx_v)
    plsc.subcore_barrier()

    pltpu.async_copy(u_hbm.at[sidx_v.at[0]], rows_v.at[0], sem.at[0])
    pltpu.async_copy(u_hbm.at[sidx_v.at[1]], rows_v.at[1], sem.at[1])

    def chunk(j, carry):
        b = lax.rem(j, 2)
        pltpu.make_async_copy(u_hbm.at[sidx_v.at[j]], rows_v.at[b],
                              sem.at[b]).wait()
        pltpu.sync_copy(rows_v.at[b], agg_sp.at[didx_v.at[j]], add=True)

        @pl.when(j + 2 < NITER)
        def _():
            pltpu.async_copy(u_hbm.at[sidx_v.at[j + 2]], rows_v.at[b],
                             sem.at[b])

        return carry

    lax.fori_loop(0, NITER, chunk, 0, unroll=False)
    plsc.subcore_barrier()
    pltpu.sync_copy(agg_sp.at[pl.ds(s * ROWS_PT, ROWS_PT)],
                    out_hbm.at[c, pl.ds(s * ROWS_PT, ROWS_PT)])


def _scatter_partials(src_r, dst_r, u, zeros_d, d):
    return pl.kernel(
        _scatter_body,
        out_type=jax.ShapeDtypeStruct((NC, N_PAD, d), jnp.float32),
        mesh=_mesh,
        scratch_types=[
            pltpu.VMEM_SHARED((N_PAD, d), jnp.float32),
            pltpu.VMEM((NITER, K), jnp.int32),
            pltpu.VMEM((NITER, K), jnp.int32),
            pltpu.VMEM((2, K, d), jnp.float32),
            pltpu.SemaphoreType.DMA((2,)),
        ],
        compiler_params=pltpu.CompilerParams(use_tc_tiling_on_sc=False),
    )(src_r, dst_r, u, zeros_d)


_RB = 1000


def _dinv_of(degp_ref):
    deg = degp_ref[0, :, 0:1] + degp_ref[1, :, 0:1] + 1.0
    return lax.rsqrt(deg)


def _mm1_body(degp_ref, x_ref, w_ref, u_ref):
    dinv = _dinv_of(degp_ref)
    xw = jnp.dot(x_ref[...], w_ref[...], preferred_element_type=jnp.float32)
    u_ref[...] = dinv * xw


def _mid_body(degp_ref, agg_ref, u_ref, b_ref, w_ref, u2_ref):
    dinv = _dinv_of(degp_ref)
    tot = agg_ref[0] + agg_ref[1] + u_ref[...]
    h = jnp.maximum(dinv * tot + b_ref[...], 0.0)
    hw = jnp.dot(h, w_ref[...], preferred_element_type=jnp.float32)
    u2_ref[...] = dinv * hw


def _fin_body(degp_ref, agg_ref, u_ref, b_ref, out_ref):
    dinv = _dinv_of(degp_ref)
    out_ref[...] = dinv * (agg_ref[0] + agg_ref[1] + u_ref[...]) + b_ref[...]


def _deg_spec():
    return pl.BlockSpec((NC, _RB, DEGW), lambda i: (0, i, 0))


def _full(shape):
    return pl.BlockSpec(shape, lambda i: tuple(0 for _ in shape))


def _tc_mm1(degp, x, w1):
    return pl.pallas_call(
        _mm1_body,
        grid=(N_NODES // _RB,),
        in_specs=[_deg_spec(),
                  pl.BlockSpec((_RB, 128), lambda i: (i, 0)),
                  _full((128, 128))],
        out_specs=pl.BlockSpec((_RB, 128), lambda i: (i, 0)),
        out_shape=jax.ShapeDtypeStruct((N_NODES, 128), jnp.float32),
    )(degp, x, w1)


def _tc_mid(degp, agg1, u1, b1, w2):
    return pl.pallas_call(
        _mid_body,
        grid=(N_NODES // _RB,),
        in_specs=[_deg_spec(),
                  pl.BlockSpec((NC, _RB, 128), lambda i: (0, i, 0)),
                  pl.BlockSpec((_RB, 128), lambda i: (i, 0)),
                  _full((1, 128)),
                  _full((128, 64))],
        out_specs=pl.BlockSpec((_RB, 64), lambda i: (i, 0)),
        out_shape=jax.ShapeDtypeStruct((N_NODES, 64), jnp.float32),
    )(degp, agg1, u1, b1, w2)


def _tc_fin(degp, agg2, u2, b2):
    return pl.pallas_call(
        _fin_body,
        grid=(N_NODES // _RB,),
        in_specs=[_deg_spec(),
                  pl.BlockSpec((NC, _RB, 64), lambda i: (0, i, 0)),
                  pl.BlockSpec((_RB, 64), lambda i: (i, 0)),
                  _full((1, 64))],
        out_specs=pl.BlockSpec((_RB, 64), lambda i: (i, 0)),
        out_shape=jax.ShapeDtypeStruct((N_NODES, 64), jnp.float32),
    )(degp, agg2, u2, b2)


def kernel(x, edge_index, W1, b1, W2, b2):
    npad = E_PAD - N_EDGES
    pad_iota = jnp.arange(npad, dtype=jnp.int32)
    pad_src = (pad_iota * 7) % N_NODES
    pad_dst = N_NODES + pad_iota % (N_PAD - N_NODES)
    src_r = jnp.concatenate(
        [edge_index[0].astype(jnp.int32), pad_src]).reshape(NW, NITER, K)
    dst_r = jnp.concatenate(
        [edge_index[1].astype(jnp.int32), pad_dst]).reshape(NW, NITER, K)
    ones16 = jnp.ones((K, DEGW), jnp.float32)
    zeros16 = jnp.zeros((N_PAD, DEGW), jnp.float32)
    zeros128 = jnp.zeros((N_PAD, 128), jnp.float32)
    zeros64 = jnp.zeros((N_PAD, 64), jnp.float32)

    degp = _degree_partials(dst_r, ones16, zeros16)
    u1 = _tc_mm1(degp, x, W1)
    agg1 = _scatter_partials(src_r, dst_r, u1, zeros128, 128)
    u2 = _tc_mid(degp, agg1, u1, b1.reshape(1, 128), W2)
    agg2 = _scatter_partials(src_r, dst_r, u2, zeros64, 64)
    return _tc_fin(degp, agg2, u2, b2.reshape(1, 64))


# speedup vs baseline: 1.7083x; 1.0008x over previous
import jax
import jax.numpy as jnp
from jax import lax
from jax.experimental import pallas as pl
from jax.experimental.pallas import tpu as pltpu
from jax.experimental.pallas import tpu_sc as plsc

N_NODES = 10000
N_PAD = 10240
N_EDGES = 320000
NC = 2
NS = 16
NW = NC * NS
K = 112
NITER = 90
E_PAD = NW * NITER * K
ROWS_PT = N_PAD // NS
DEGW = 16

_mesh = plsc.VectorSubcoreMesh(core_axis_name="c", subcore_axis_name="s")


def _deg_body(dst_hbm, ones_hbm, zeros_hbm, out_hbm, deg_sp, idx_v, ones_v):
    c = lax.axis_index("c")
    s = lax.axis_index("s")
    wid = s * NC + c
    pltpu.sync_copy(zeros_hbm.at[pl.ds(s * ROWS_PT, ROWS_PT)],
                    deg_sp.at[pl.ds(s * ROWS_PT, ROWS_PT)])
    pltpu.sync_copy(ones_hbm, ones_v)
    pltpu.sync_copy(dst_hbm.at[wid], idx_v)
    plsc.subcore_barrier()

    def chunk(j, carry):
        pltpu.sync_copy(ones_v, deg_sp.at[idx_v.at[j]], add=True)
        return carry

    lax.fori_loop(0, NITER, chunk, 0, unroll=False)
    plsc.subcore_barrier()
    pltpu.sync_copy(deg_sp.at[pl.ds(s * ROWS_PT, ROWS_PT)],
                    out_hbm.at[c, pl.ds(s * ROWS_PT, ROWS_PT)])


def _degree_partials(dst_r, ones16, zeros16):
    return pl.kernel(
        _deg_body,
        out_type=jax.ShapeDtypeStruct((NC, N_PAD, DEGW), jnp.float32),
        mesh=_mesh,
        scratch_types=[
            pltpu.VMEM_SHARED((N_PAD, DEGW), jnp.float32),
            pltpu.VMEM((NITER, K), jnp.int32),
            pltpu.VMEM((K, DEGW), jnp.float32),
        ],
        compiler_params=pltpu.CompilerParams(use_tc_tiling_on_sc=False),
    )(dst_r, ones16, zeros16)


def _scatter_body(src_hbm, dst_hbm, u_hbm, zeros_hbm, out_hbm,
                  agg_sp, sidx_v, didx_v, rows_v, sem):
    c = lax.axis_index("c")
    s = lax.axis_index("s")
    wid = s * NC + c
    pltpu.sync_copy(zeros_hbm.at[pl.ds(s * ROWS_PT, ROWS_PT)],
                    agg_sp.at[pl.ds(s * ROWS_PT, ROWS_PT)])
    pltpu.sync_copy(src_hbm.at[wid], sidx_v)
    pltpu.sync_copy(dst_hbm.at[wid], didx_v)
    plsc.subcore_barrier()

    pltpu.async_copy(u_hbm.at[sidx_v.at[0]], rows_v.at[0], sem.at[0])
    pltpu.async_copy(u_hbm.at[sidx_v.at[1]], rows_v.at[1], sem.at[1])

    def chunk(j, carry):
        b = lax.rem(j, 2)
        pltpu.make_async_copy(u_hbm.at[sidx_v.at[j]], rows_v.at[b],
                              sem.at[b]).wait()
        pltpu.sync_copy(rows_v.at[b], agg_sp.at[didx_v.at[j]], add=True)

        @pl.when(j + 2 < NITER)
        def _():
            pltpu.async_copy(u_hbm.at[sidx_v.at[j + 2]], rows_v.at[b],
                             sem.at[b])

        return carry

    lax.fori_loop(0, NITER, chunk, 0, unroll=False)
    plsc.subcore_barrier()
    pltpu.sync_copy(agg_sp.at[pl.ds(s * ROWS_PT, ROWS_PT)],
                    out_hbm.at[c, pl.ds(s * ROWS_PT, ROWS_PT)])


def _scatter_partials(src_r, dst_r, u, zeros_d, d):
    return pl.kernel(
        _scatter_body,
        out_type=jax.ShapeDtypeStruct((NC, N_PAD, d), jnp.float32),
        mesh=_mesh,
        scratch_types=[
            pltpu.VMEM_SHARED((N_PAD, d), jnp.float32),
            pltpu.VMEM((NITER, K), jnp.int32),
            pltpu.VMEM((NITER, K), jnp.int32),
            pltpu.VMEM((2, K, d), jnp.float32),
            pltpu.SemaphoreType.DMA((2,)),
        ],
        compiler_params=pltpu.CompilerParams(use_tc_tiling_on_sc=False),
    )(src_r, dst_r, u, zeros_d)


_RB = 1000


def _dinv_of(degp_ref):
    deg = degp_ref[0, :, 0:1] + degp_ref[1, :, 0:1] + 1.0
    return lax.rsqrt(deg)


def _mm1_body(degp_ref, x_ref, w_ref, u_ref):
    dinv = _dinv_of(degp_ref)
    xw = jnp.dot(x_ref[...], w_ref[...], preferred_element_type=jnp.float32)
    u_ref[...] = dinv * xw


def _mid_body(degp_ref, agg_ref, u_ref, b_ref, w_ref, u2_ref):
    dinv = _dinv_of(degp_ref)
    tot = agg_ref[0] + agg_ref[1] + u_ref[...]
    h = jnp.maximum(dinv * tot + b_ref[...], 0.0)
    hw = jnp.dot(h, w_ref[...], preferred_element_type=jnp.float32)
    u2_ref[...] = dinv * hw


def _fin_body(degp_ref, agg_ref, u_ref, b_ref, out_ref):
    dinv = _dinv_of(degp_ref)
    out_ref[...] = dinv * (agg_ref[0] + agg_ref[1] + u_ref[...]) + b_ref[...]


def _deg_spec():
    return pl.BlockSpec((NC, _RB, DEGW), lambda i: (0, i, 0))


def _full(shape):
    return pl.BlockSpec(shape, lambda i: tuple(0 for _ in shape))


def _tc_mm1(degp, x, w1):
    return pl.pallas_call(
        _mm1_body,
        grid=(N_NODES // _RB,),
        in_specs=[_deg_spec(),
                  pl.BlockSpec((_RB, 128), lambda i: (i, 0)),
                  _full((128, 128))],
        out_specs=pl.BlockSpec((_RB, 128), lambda i: (i, 0)),
        out_shape=jax.ShapeDtypeStruct((N_NODES, 128), jnp.float32),
    )(degp, x, w1)


def _tc_mid(degp, agg1, u1, b1, w2):
    return pl.pallas_call(
        _mid_body,
        grid=(N_NODES // _RB,),
        in_specs=[_deg_spec(),
                  pl.BlockSpec((NC, _RB, 128), lambda i: (0, i, 0)),
                  pl.BlockSpec((_RB, 128), lambda i: (i, 0)),
                  _full((1, 128)),
                  _full((128, 64))],
        out_specs=pl.BlockSpec((_RB, 64), lambda i: (i, 0)),
        out_shape=jax.ShapeDtypeStruct((N_NODES, 64), jnp.float32),
    )(degp, agg1, u1, b1, w2)


def _tc_fin(degp, agg2, u2, b2):
    return pl.pallas_call(
        _fin_body,
        grid=(N_NODES // _RB,),
        in_specs=[_deg_spec(),
                  pl.BlockSpec((NC, _RB, 64), lambda i: (0, i, 0)),
                  pl.BlockSpec((_RB, 64), lambda i: (i, 0)),
                  _full((1, 64))],
        out_specs=pl.BlockSpec((_RB, 64), lambda i: (i, 0)),
        out_shape=jax.ShapeDtypeStruct((N_NODES, 64), jnp.float32),
    )(degp, agg2, u2, b2)


def kernel(x, edge_index, W1, b1, W2, b2):
    npad = E_PAD - N_EDGES
    pad_iota = jnp.arange(npad, dtype=jnp.int32)
    pad_src = (pad_iota * 7) % N_NODES
    pad_dst = N_NODES + pad_iota % (N_PAD - N_NODES)
    src_r = jnp.concatenate(
        [edge_index[0].astype(jnp.int32), pad_src]).reshape(NW, NITER, K)
    dst_r = jnp.concatenate(
        [edge_index[1].astype(jnp.int32), pad_dst]).reshape(NW, NITER, K)
    ones16 = jnp.ones((K, DEGW), jnp.float32)
    zeros16 = jnp.zeros((N_PAD, DEGW), jnp.float32)
    zeros128 = jnp.zeros((N_PAD, 128), jnp.float32)
    zeros64 = jnp.zeros((N_PAD, 64), jnp.float32)

    degp = _degree_partials(dst_r, ones16, zeros16)
    u1 = _tc_mm1(degp, x, W1)
    agg1 = _scatter_partials(src_r, dst_r, u1, zeros128, 128)
    u2 = _tc_mid(degp, agg1, u1, b1.reshape(1, 128), W2)
    agg2 = _scatter_partials(src_r, dst_r, u2, zeros64, 64)
    return _tc_fin(degp, agg2, u2, b2.reshape(1, 64))
